# baseline (speedup 1.0000x reference)
import functools

import jax
import jax.numpy as jnp
from jax import lax
from jax.experimental import pallas as pl
from jax.experimental.pallas import tpu as pltpu
from jax.experimental.pallas import tpu_sc as plsc

_N_ATTR = 26
_N_VALUES = 1000
_EMBED_DIM = 128
_N_HIDDEN = 1024
_BATCH = 4096
_K = _N_ATTR * _EMBED_DIM

_NC = 2
_NS = 16
_NW = _NC * _NS

_CH = 128
_NCHUNK = 2
_BC = _BATCH // _NCHUNK


@functools.cache
def _build_gather_sc(bc):
    rows = bc * _N_ATTR
    rpw = rows // _NW
    nch = rpw // _CH
    mesh = plsc.VectorSubcoreMesh(
        core_axis_name="c", subcore_axis_name="s", num_cores=_NC, num_subcores=_NS
    )

    @functools.partial(
        pl.kernel,
        out_type=jax.ShapeDtypeStruct((bc, _K), jnp.float32),
        mesh=mesh,
        scratch_types=[
            pltpu.VMEM((nch, _CH), jnp.int32),
            pltpu.VMEM((2, _CH, _EMBED_DIM), jnp.float32),
            pltpu.SemaphoreType.DMA,
            pltpu.SemaphoreType.DMA,
            pltpu.SemaphoreType.DMA,
            pltpu.SemaphoreType.DMA,
        ],
    )
    def _gather_sc(idx_hbm, table_hbm, out_hbm, idx_v, rows_v, g0, g1, w0, w1):
        wid = lax.axis_index("s") * _NC + lax.axis_index("c")
        pltpu.sync_copy(idx_hbm.at[wid], idx_v)
        gsems = (g0, g1)
        wsems = (w0, w1)
        base = wid * rpw
        gathers = [None] * nch
        writes = [None] * nch
        gathers[0] = pltpu.async_copy(table_hbm.at[idx_v.at[0]], rows_v.at[0], gsems[0])
        for j in range(nch):
            b = j & 1
            gathers[j].wait()
            if j >= 1:
                writes[j - 1].wait()
            if j + 1 < nch:
                gathers[j + 1] = pltpu.async_copy(
                    table_hbm.at[idx_v.at[j + 1]], rows_v.at[b ^ 1], gsems[b ^ 1]
                )
            r0 = base + j * _CH
            writes[j] = pltpu.async_copy(
                rows_v.at[b],
                out_hbm.at[pl.ds(r0 % bc, _CH), pl.ds((r0 // bc) * _EMBED_DIM, _EMBED_DIM)],
                wsems[b],
            )
        writes[nch - 1].wait()

    return _gather_sc


_BM = 512


def _mm_body(a_ref, w_ref, b_ref, o_ref):
    o_ref[...] = (
        jnp.dot(a_ref[...], w_ref[...], preferred_element_type=jnp.float32)
        + b_ref[...]
    )


def _mm_body_alias(a_ref, w_ref, b_ref, prev_ref, o_ref):
    del prev_ref
    o_ref[...] = (
        jnp.dot(a_ref[...], w_ref[...], preferred_element_type=jnp.float32)
        + b_ref[...]
    )


def _matmul_tc(flat, fc_w, fc_b2d, prev, c):
    base = c * (_BC // _BM)
    return pl.pallas_call(
        _mm_body_alias,
        grid=(_BC // _BM,),
        in_specs=[
            pl.BlockSpec((_BM, _K), lambda i: (i, 0)),
            pl.BlockSpec((_K, _N_HIDDEN), lambda i: (0, 0)),
            pl.BlockSpec((1, _N_HIDDEN), lambda i: (0, 0)),
            pl.BlockSpec(memory_space=pl.ANY),
        ],
        out_specs=pl.BlockSpec((_BM, _N_HIDDEN), lambda i, base=base: (base + i, 0)),
        out_shape=jax.ShapeDtypeStruct((_BATCH, _N_HIDDEN), jnp.float32),
        input_output_aliases={3: 0},
    )(flat, fc_w, fc_b2d, prev)


def kernel(x, table, fc_w, fc_b):
    offs = (jnp.arange(_N_ATTR, dtype=jnp.int32) * _N_VALUES)[:, None]
    xT = x.astype(jnp.int32).T + offs
    gather = _build_gather_sc(_BC)
    nch = _BC * _N_ATTR // (_NW * _CH)
    flats = []
    for c in range(_NCHUNK):
        idx_c = lax.slice_in_dim(xT, c * _BC, (c + 1) * _BC, axis=1)
        flats.append(gather(idx_c.reshape(_NW, nch, _CH), table))
    fc_b2d = fc_b.reshape(1, _N_HIDDEN)
    out = jnp.zeros((_BATCH, _N_HIDDEN), jnp.float32)
    for c in range(_NCHUNK):
        out = _matmul_tc(flats[c], fc_w, fc_b2d, out, c)
    return out


# speedup vs baseline: 1.0798x; 1.0798x over previous
import functools

import jax
import jax.numpy as jnp
from jax import lax
from jax.experimental import pallas as pl
from jax.experimental.pallas import tpu as pltpu
from jax.experimental.pallas import tpu_sc as plsc

_N_ATTR = 26
_N_VALUES = 1000
_EMBED_DIM = 128
_N_HIDDEN = 1024
_BATCH = 4096
_K = _N_ATTR * _EMBED_DIM

_NC = 2
_NS = 16
_NW = _NC * _NS

_CH = 128
_NCHUNK = 2
_BC = _BATCH // _NCHUNK


@functools.cache
def _build_gather_sc(bc):
    rows = bc * _N_ATTR
    rpw = rows // _NW
    nch = rpw // _CH
    mesh = plsc.VectorSubcoreMesh(
        core_axis_name="c", subcore_axis_name="s", num_cores=_NC, num_subcores=_NS
    )

    @functools.partial(
        pl.kernel,
        out_type=jax.ShapeDtypeStruct((bc, _K), jnp.float32),
        mesh=mesh,
        scratch_types=[
            pltpu.VMEM((nch, _CH), jnp.int32),
            pltpu.VMEM((2, _CH, _EMBED_DIM), jnp.float32),
            pltpu.SemaphoreType.DMA,
            pltpu.SemaphoreType.DMA,
            pltpu.SemaphoreType.DMA,
            pltpu.SemaphoreType.DMA,
        ],
    )
    def _gather_sc(idx_hbm, table_hbm, out_hbm, idx_v, rows_v, g0, g1, w0, w1):
        wid = lax.axis_index("s") * _NC + lax.axis_index("c")
        pltpu.sync_copy(idx_hbm.at[wid], idx_v)
        gsems = (g0, g1)
        wsems = (w0, w1)
        base = wid * rpw
        gathers = [None] * nch
        writes = [None] * nch
        gathers[0] = pltpu.async_copy(table_hbm.at[idx_v.at[0]], rows_v.at[0], gsems[0])
        for j in range(nch):
            b = j & 1
            gathers[j].wait()
            if j >= 1:
                writes[j - 1].wait()
            if j + 1 < nch:
                gathers[j + 1] = pltpu.async_copy(
                    table_hbm.at[idx_v.at[j + 1]], rows_v.at[b ^ 1], gsems[b ^ 1]
                )
            r0 = base + j * _CH
            writes[j] = pltpu.async_copy(
                rows_v.at[b],
                out_hbm.at[pl.ds(r0 % bc, _CH), pl.ds((r0 // bc) * _EMBED_DIM, _EMBED_DIM)],
                wsems[b],
            )
        writes[nch - 1].wait()

    return _gather_sc


_BM = 512


def _mm_body(a_ref, w_ref, b_ref, o_ref):
    o_ref[...] = (
        jnp.dot(a_ref[...], w_ref[...], preferred_element_type=jnp.float32)
        + b_ref[...]
    )


def _mm_body_alias(a_ref, w_ref, b_ref, prev_ref, o_ref):
    del prev_ref
    o_ref[...] = (
        jnp.dot(a_ref[...], w_ref[...], preferred_element_type=jnp.float32)
        + b_ref[...]
    )


def _matmul_tc(flat, fc_w, fc_b2d, prev, c):
    base = c * (_BC // _BM)
    out_spec = pl.BlockSpec((_BM, _N_HIDDEN), lambda i, base=base: (base + i, 0))
    in_specs = [
        pl.BlockSpec((_BM, _K), lambda i: (i, 0)),
        pl.BlockSpec((_K, _N_HIDDEN), lambda i: (0, 0)),
        pl.BlockSpec((1, _N_HIDDEN), lambda i: (0, 0)),
    ]
    args = [flat, fc_w, fc_b2d]
    if prev is None:
        body, aliases = _mm_body, {}
    else:
        body, aliases = _mm_body_alias, {3: 0}
        in_specs.append(pl.BlockSpec(memory_space=pl.ANY))
        args.append(prev)
    return pl.pallas_call(
        body,
        grid=(_BC // _BM,),
        in_specs=in_specs,
        out_specs=out_spec,
        out_shape=jax.ShapeDtypeStruct((_BATCH, _N_HIDDEN), jnp.float32),
        input_output_aliases=aliases,
    )(*args)


def kernel(x, table, fc_w, fc_b):
    offs = (jnp.arange(_N_ATTR, dtype=jnp.int32) * _N_VALUES)[:, None]
    xT = x.astype(jnp.int32).T + offs
    gather = _build_gather_sc(_BC)
    nch = _BC * _N_ATTR // (_NW * _CH)
    flats = []
    for c in range(_NCHUNK):
        idx_c = lax.slice_in_dim(xT, c * _BC, (c + 1) * _BC, axis=1)
        flats.append(gather(idx_c.reshape(_NW, nch, _CH), table))
    fc_b2d = fc_b.reshape(1, _N_HIDDEN)
    out = None
    for c in range(_NCHUNK):
        out = _matmul_tc(flats[c], fc_w, fc_b2d, out, c)
    return out


# speedup vs baseline: 1.1671x; 1.0809x over previous
import functools

import jax
import jax.numpy as jnp
from jax import lax
from jax.experimental import pallas as pl
from jax.experimental.pallas import tpu as pltpu
from jax.experimental.pallas import tpu_sc as plsc

_N_ATTR = 26
_N_VALUES = 1000
_EMBED_DIM = 128
_N_HIDDEN = 1024
_BATCH = 4096
_K = _N_ATTR * _EMBED_DIM

_NC = 2
_NS = 16
_NW = _NC * _NS

_CH = 128
_NBUF = 4
_NCHUNK = 2
_BC = _BATCH // _NCHUNK


@functools.cache
def _build_gather_sc(bc):
    rows = bc * _N_ATTR
    rpw = rows // _NW
    nch = rpw // _CH
    mesh = plsc.VectorSubcoreMesh(
        core_axis_name="c", subcore_axis_name="s", num_cores=_NC, num_subcores=_NS
    )

    @functools.partial(
        pl.kernel,
        out_type=jax.ShapeDtypeStruct((bc, _K), jnp.float32),
        mesh=mesh,
        scratch_types=[
            pltpu.VMEM((nch, _CH), jnp.int32),
            pltpu.VMEM((_NBUF, _CH, _EMBED_DIM), jnp.float32),
            [pltpu.SemaphoreType.DMA] * _NBUF,
            [pltpu.SemaphoreType.DMA] * _NBUF,
        ],
    )
    def _gather_sc(idx_hbm, table_hbm, out_hbm, idx_v, rows_v, gsems, wsems):
        wid = lax.axis_index("s") * _NC + lax.axis_index("c")
        pltpu.sync_copy(idx_hbm.at[wid], idx_v)
        base = wid * rpw
        gathers = [None] * nch
        writes = [None] * nch

        def _gather(j):
            return pltpu.async_copy(
                table_hbm.at[idx_v.at[j]], rows_v.at[j % _NBUF], gsems[j % _NBUF]
            )

        for j in range(min(_NBUF - 1, nch)):
            gathers[j] = _gather(j)
        for j in range(nch):
            b = j % _NBUF
            gathers[j].wait()
            jn = j + _NBUF - 1
            if jn < nch:
                if jn >= _NBUF:
                    writes[jn - _NBUF].wait()
                gathers[jn] = _gather(jn)
            r0 = base + j * _CH
            writes[j] = pltpu.async_copy(
                rows_v.at[b],
                out_hbm.at[pl.ds(r0 % bc, _CH), pl.ds((r0 // bc) * _EMBED_DIM, _EMBED_DIM)],
                wsems[b],
            )
        for j in range(max(0, nch - _NBUF), nch):
            writes[j].wait()

    return _gather_sc


_BM = 512


def _mm_body(a_ref, w_ref, b_ref, o_ref):
    o_ref[...] = (
        jnp.dot(a_ref[...], w_ref[...], preferred_element_type=jnp.float32)
        + b_ref[...]
    )


def _mm_body_alias(a_ref, w_ref, b_ref, prev_ref, o_ref):
    del prev_ref
    o_ref[...] = (
        jnp.dot(a_ref[...], w_ref[...], preferred_element_type=jnp.float32)
        + b_ref[...]
    )


def _matmul_tc(flat, fc_w, fc_b2d, prev, c):
    base = c * (_BC // _BM)
    out_spec = pl.BlockSpec((_BM, _N_HIDDEN), lambda i, base=base: (base + i, 0))
    in_specs = [
        pl.BlockSpec((_BM, _K), lambda i: (i, 0)),
        pl.BlockSpec((_K, _N_HIDDEN), lambda i: (0, 0)),
        pl.BlockSpec((1, _N_HIDDEN), lambda i: (0, 0)),
    ]
    args = [flat, fc_w, fc_b2d]
    if prev is None:
        body, aliases = _mm_body, {}
    else:
        body, aliases = _mm_body_alias, {3: 0}
        in_specs.append(pl.BlockSpec(memory_space=pl.ANY))
        args.append(prev)
    return pl.pallas_call(
        body,
        grid=(_BC // _BM,),
        in_specs=in_specs,
        out_specs=out_spec,
        out_shape=jax.ShapeDtypeStruct((_BATCH, _N_HIDDEN), jnp.float32),
        input_output_aliases=aliases,
    )(*args)


def kernel(x, table, fc_w, fc_b):
    offs = (jnp.arange(_N_ATTR, dtype=jnp.int32) * _N_VALUES)[:, None]
    xT = x.astype(jnp.int32).T + offs
    gather = _build_gather_sc(_BC)
    nch = _BC * _N_ATTR // (_NW * _CH)
    flats = []
    for c in range(_NCHUNK):
        idx_c = lax.slice_in_dim(xT, c * _BC, (c + 1) * _BC, axis=1)
        flats.append(gather(idx_c.reshape(_NW, nch, _CH), table))
    fc_b2d = fc_b.reshape(1, _N_HIDDEN)
    out = None
    for c in range(_NCHUNK):
        out = _matmul_tc(flats[c], fc_w, fc_b2d, out, c)
    return out


# speedup vs baseline: 1.1847x; 1.0150x over previous
import functools

import jax
import jax.numpy as jnp
from jax import lax
from jax.experimental import pallas as pl
from jax.experimental.pallas import tpu as pltpu
from jax.experimental.pallas import tpu_sc as plsc

_N_ATTR = 26
_N_VALUES = 1000
_EMBED_DIM = 128
_N_HIDDEN = 1024
_BATCH = 4096
_K = _N_ATTR * _EMBED_DIM

_NC = 2
_NS = 16
_NW = _NC * _NS

_CH = 128
_NBUF = 6
_NCHUNK = 2
_BC = _BATCH // _NCHUNK


@functools.cache
def _build_gather_sc(bc):
    rows = bc * _N_ATTR
    rpw = rows // _NW
    nch = rpw // _CH
    mesh = plsc.VectorSubcoreMesh(
        core_axis_name="c", subcore_axis_name="s", num_cores=_NC, num_subcores=_NS
    )

    @functools.partial(
        pl.kernel,
        out_type=jax.ShapeDtypeStruct((bc, _K), jnp.float32),
        mesh=mesh,
        scratch_types=[
            pltpu.VMEM((nch, _CH), jnp.int32),
            pltpu.VMEM((_NBUF, _CH, _EMBED_DIM), jnp.float32),
            [pltpu.SemaphoreType.DMA] * _NBUF,
            [pltpu.SemaphoreType.DMA] * _NBUF,
        ],
    )
    def _gather_sc(idx_hbm, table_hbm, out_hbm, idx_v, rows_v, gsems, wsems):
        wid = lax.axis_index("s") * _NC + lax.axis_index("c")
        pltpu.sync_copy(idx_hbm.at[wid], idx_v)
        base = wid * rpw
        gathers = [None] * nch
        writes = [None] * nch

        def _gather(j):
            return pltpu.async_copy(
                table_hbm.at[idx_v.at[j]], rows_v.at[j % _NBUF], gsems[j % _NBUF]
            )

        for j in range(min(_NBUF - 1, nch)):
            gathers[j] = _gather(j)
        for j in range(nch):
            b = j % _NBUF
            gathers[j].wait()
            jn = j + _NBUF - 1
            if jn < nch:
                if jn >= _NBUF:
                    writes[jn - _NBUF].wait()
                gathers[jn] = _gather(jn)
            r0 = base + j * _CH
            writes[j] = pltpu.async_copy(
                rows_v.at[b],
                out_hbm.at[pl.ds(r0 % bc, _CH), pl.ds((r0 // bc) * _EMBED_DIM, _EMBED_DIM)],
                wsems[b],
            )
        for j in range(max(0, nch - _NBUF), nch):
            writes[j].wait()

    return _gather_sc


_BM = 512


def _mm_body(a_ref, w_ref, b_ref, o_ref):
    o_ref[...] = (
        jnp.dot(a_ref[...], w_ref[...], preferred_element_type=jnp.float32)
        + b_ref[...]
    )


def _mm_body_alias(a_ref, w_ref, b_ref, prev_ref, o_ref):
    del prev_ref
    o_ref[...] = (
        jnp.dot(a_ref[...], w_ref[...], preferred_element_type=jnp.float32)
        + b_ref[...]
    )


def _matmul_tc(flat, fc_w, fc_b2d, prev, c):
    base = c * (_BC // _BM)
    out_spec = pl.BlockSpec((_BM, _N_HIDDEN), lambda i, base=base: (base + i, 0))
    in_specs = [
        pl.BlockSpec((_BM, _K), lambda i: (i, 0)),
        pl.BlockSpec((_K, _N_HIDDEN), lambda i: (0, 0)),
        pl.BlockSpec((1, _N_HIDDEN), lambda i: (0, 0)),
    ]
    args = [flat, fc_w, fc_b2d]
    if prev is None:
        body, aliases = _mm_body, {}
    else:
        body, aliases = _mm_body_alias, {3: 0}
        in_specs.append(pl.BlockSpec(memory_space=pl.ANY))
        args.append(prev)
    return pl.pallas_call(
        body,
        grid=(_BC // _BM,),
        in_specs=in_specs,
        out_specs=out_spec,
        out_shape=jax.ShapeDtypeStruct((_BATCH, _N_HIDDEN), jnp.float32),
        input_output_aliases=aliases,
    )(*args)


def kernel(x, table, fc_w, fc_b):
    offs = (jnp.arange(_N_ATTR, dtype=jnp.int32) * _N_VALUES)[:, None]
    xT = x.astype(jnp.int32).T + offs
    gather = _build_gather_sc(_BC)
    nch = _BC * _N_ATTR // (_NW * _CH)
    flats = []
    for c in range(_NCHUNK):
        idx_c = lax.slice_in_dim(xT, c * _BC, (c + 1) * _BC, axis=1)
        flats.append(gather(idx_c.reshape(_NW, nch, _CH), table))
    fc_b2d = fc_b.reshape(1, _N_HIDDEN)
    out = None
    for c in range(_NCHUNK):
        out = _matmul_tc(flats[c], fc_w, fc_b2d, out, c)
    return out
